# baseline (speedup 1.0000x reference)
import jax
import jax.numpy as jnp
from jax.experimental import pallas as pl

_BETA = 0.25
_TB = 128
_WIN = 4096
_KCAND = 4


def _exact_d(cand_idx, xbf, wb32, w_f32, xsq, wsq, K):
    TB = xbf.shape[0]
    onehot = (jax.lax.broadcasted_iota(jnp.int32, (TB, K), 1)
              == cand_idx[:, None]).astype(jnp.float32)
    wrow = jax.lax.dot_general(onehot, wb32, (((1,), (0,)), ((), ())),
                               preferred_element_type=jnp.float32,
                               precision=jax.lax.Precision.HIGHEST)
    wsq_c = jax.lax.dot_general(onehot, wsq[:, None], (((1,), (0,)), ((), ())),
                                preferred_element_type=jnp.float32,
                                precision=jax.lax.Precision.HIGHEST)[:, 0]
    prods = xbf * wrow
    s = prods[:, 0]
    c = jnp.zeros_like(s)
    for k in range(1, prods.shape[1]):
        p = prods[:, k]
        t = s + p
        e = jnp.where(jnp.abs(s) >= jnp.abs(p), (s - t) + p, (p - t) + s)
        c = c + e
        s = t
    m = s + c
    return (xsq + wsq_c) - 2.0 * m


def _window_winner(dwin, base, xbf, wb32, w_f32, xsq, wsq, K):
    TB = dwin.shape[0]
    cur = dwin
    iota = jax.lax.broadcasted_iota(jnp.int32, dwin.shape, 1)
    best_v = None
    best_i = None
    for _ in range(_KCAND):
        i_r = jnp.argmin(cur, axis=1).astype(jnp.int32)
        cur = jnp.where(iota == i_r[:, None], jnp.float32(jnp.inf), cur)
        gi = i_r + base
        gv = _exact_d(gi, xbf, wb32, w_f32, xsq, wsq, K)
        if best_v is None:
            best_v, best_i = gv, gi
        else:
            better = (gv < best_v) | ((gv == best_v) & (gi < best_i))
            best_v = jnp.where(better, gv, best_v)
            best_i = jnp.where(better, gi, best_i)
    return best_v, best_i


def _vq_body(x_ref, xb_ref, w_ref, wb_ref, xsq_ref, wsq_ref,
             xqst_ref, idx_ref, loss_ref):
    i = pl.program_id(0)
    K = w_ref.shape[0]
    x_blk = x_ref[...]
    xb = xb_ref[...]
    w = w_ref[...]
    wb = wb_ref[...]
    xsq = xsq_ref[...]
    wsq = wsq_ref[...]
    xbf = xb.astype(jnp.float32)
    wb32 = wb.astype(jnp.float32)
    m = jax.lax.dot_general(xb, wb, (((1,), (1,)), ((), ())),
                            preferred_element_type=jnp.float32)
    d = (xsq[:, None] + wsq[None, :]) - 2.0 * m
    v1, i1 = _window_winner(d[:, :_WIN], 0, xbf, wb32, w, xsq, wsq, K)
    v2, i2 = _window_winner(d[:, _WIN:], _WIN, xbf, wb32, w, xsq, wsq, K)
    v1q = v1.astype(jnp.bfloat16).astype(jnp.float32)
    idx = jnp.where(v2 < v1q, i2, i1)
    onehot = (jax.lax.broadcasted_iota(jnp.int32, (x_blk.shape[0], K), 1)
              == idx[:, None]).astype(jnp.float32)
    xq = jax.lax.dot_general(onehot, w, (((1,), (0,)), ((), ())),
                             preferred_element_type=jnp.float32,
                             precision=jax.lax.Precision.HIGHEST)
    diff = xq - x_blk
    xqst_ref[...] = x_blk + diff
    idx_ref[...] = idx

    @pl.when(i == 0)
    def _():
        loss_ref[...] = jnp.zeros((1, 1), jnp.float32)

    loss_ref[...] += jnp.sum(diff * diff).reshape(1, 1)


def kernel(x, W):
    B, S, D = x.shape
    T = B * S
    K = W.shape[0]
    latent = x.reshape(T, D)
    xsq = jnp.sum(latent ** 2, axis=1)
    wsq = jnp.sum(W ** 2, axis=1)
    xb = latent.astype(jnp.bfloat16)
    wb = W.astype(jnp.bfloat16)
    xqst, idx, loss_sum = pl.pallas_call(
        _vq_body,
        grid=(T // _TB,),
        in_specs=[
            pl.BlockSpec((_TB, D), lambda i: (i, 0)),
            pl.BlockSpec((_TB, D), lambda i: (i, 0)),
            pl.BlockSpec((K, D), lambda i: (0, 0)),
            pl.BlockSpec((K, D), lambda i: (0, 0)),
            pl.BlockSpec((_TB,), lambda i: (i,)),
            pl.BlockSpec((K,), lambda i: (0,)),
        ],
        out_specs=[
            pl.BlockSpec((_TB, D), lambda i: (i, 0)),
            pl.BlockSpec((_TB,), lambda i: (i,)),
            pl.BlockSpec((1, 1), lambda i: (0, 0)),
        ],
        out_shape=[
            jax.ShapeDtypeStruct((T, D), jnp.float32),
            jax.ShapeDtypeStruct((T,), jnp.int32),
            jax.ShapeDtypeStruct((1, 1), jnp.float32),
        ],
    )(latent, xb, W, wb, xsq, wsq)
    L = loss_sum[0, 0] / (T * D)
    loss = L + _BETA * L
    return xqst.reshape(x.shape), loss, idx.reshape(B, S)


# speedup vs baseline: 1.1005x; 1.1005x over previous
import jax
import jax.numpy as jnp
from jax.experimental import pallas as pl

_BETA = 0.25
_TB = 128
_WIN = 4096
_KCAND = 4


def _exact_d(cand_idx, xbf, wb32, w_f32, xsq, wsq, K):
    TB = xbf.shape[0]
    onehot = (jax.lax.broadcasted_iota(jnp.int32, (TB, K), 1)
              == cand_idx[:, None]).astype(jnp.bfloat16)
    wrow = jax.lax.dot_general(onehot, wb32.astype(jnp.bfloat16),
                               (((1,), (0,)), ((), ())),
                               preferred_element_type=jnp.float32)
    ws_hi = wsq.astype(jnp.bfloat16)
    ws_r1 = wsq - ws_hi.astype(jnp.float32)
    ws_mid = ws_r1.astype(jnp.bfloat16)
    ws_lo = (ws_r1 - ws_mid.astype(jnp.float32)).astype(jnp.bfloat16)
    wsplit = jnp.stack([ws_hi, ws_mid, ws_lo], axis=1)
    g = jax.lax.dot_general(onehot, wsplit, (((1,), (0,)), ((), ())),
                            preferred_element_type=jnp.float32)
    wsq_c = (g[:, 0] + g[:, 1]) + g[:, 2]
    prods = xbf * wrow
    s = prods[:, 0]
    c = jnp.zeros_like(s)
    for k in range(1, prods.shape[1]):
        p = prods[:, k]
        t = s + p
        e = jnp.where(jnp.abs(s) >= jnp.abs(p), (s - t) + p, (p - t) + s)
        c = c + e
        s = t
    m = s + c
    return (xsq + wsq_c) - 2.0 * m


def _window_winner(dwin, base, xbf, wb32, w_f32, xsq, wsq, K):
    TB = dwin.shape[0]
    cur = dwin
    iota = jax.lax.broadcasted_iota(jnp.int32, dwin.shape, 1)
    best_v = None
    best_i = None
    for _ in range(_KCAND):
        i_r = jnp.argmin(cur, axis=1).astype(jnp.int32)
        cur = jnp.where(iota == i_r[:, None], jnp.float32(jnp.inf), cur)
        gi = i_r + base
        gv = _exact_d(gi, xbf, wb32, w_f32, xsq, wsq, K)
        if best_v is None:
            best_v, best_i = gv, gi
        else:
            better = (gv < best_v) | ((gv == best_v) & (gi < best_i))
            best_v = jnp.where(better, gv, best_v)
            best_i = jnp.where(better, gi, best_i)
    return best_v, best_i


def _vq_body(x_ref, xb_ref, w_ref, wb_ref, xsq_ref, wsq_ref,
             xqst_ref, idx_ref, loss_ref):
    i = pl.program_id(0)
    K = w_ref.shape[0]
    x_blk = x_ref[...]
    xb = xb_ref[...]
    w = w_ref[...]
    wb = wb_ref[...]
    xsq = xsq_ref[...]
    wsq = wsq_ref[...]
    xbf = xb.astype(jnp.float32)
    wb32 = wb.astype(jnp.float32)
    m = jax.lax.dot_general(xb, wb, (((1,), (1,)), ((), ())),
                            preferred_element_type=jnp.float32)
    d = (xsq[:, None] + wsq[None, :]) - 2.0 * m
    v1, i1 = _window_winner(d[:, :_WIN], 0, xbf, wb32, w, xsq, wsq, K)
    v2, i2 = _window_winner(d[:, _WIN:], _WIN, xbf, wb32, w, xsq, wsq, K)
    v1q = v1.astype(jnp.bfloat16).astype(jnp.float32)
    idx = jnp.where(v2 < v1q, i2, i1)
    onehot = (jax.lax.broadcasted_iota(jnp.int32, (x_blk.shape[0], K), 1)
              == idx[:, None]).astype(jnp.bfloat16)
    w_hi = w.astype(jnp.bfloat16)
    w_r1 = w - w_hi.astype(jnp.float32)
    w_mid = w_r1.astype(jnp.bfloat16)
    w_lo = (w_r1 - w_mid.astype(jnp.float32)).astype(jnp.bfloat16)
    q_hi = jax.lax.dot_general(onehot, w_hi, (((1,), (0,)), ((), ())),
                               preferred_element_type=jnp.float32)
    q_mid = jax.lax.dot_general(onehot, w_mid, (((1,), (0,)), ((), ())),
                                preferred_element_type=jnp.float32)
    q_lo = jax.lax.dot_general(onehot, w_lo, (((1,), (0,)), ((), ())),
                               preferred_element_type=jnp.float32)
    xq = (q_hi + q_mid) + q_lo
    diff = xq - x_blk
    xqst_ref[...] = x_blk + diff
    idx_ref[...] = idx

    @pl.when(i == 0)
    def _():
        loss_ref[...] = jnp.zeros((1, 1), jnp.float32)

    loss_ref[...] += jnp.sum(diff * diff).reshape(1, 1)


def kernel(x, W):
    B, S, D = x.shape
    T = B * S
    K = W.shape[0]
    latent = x.reshape(T, D)
    xsq = jnp.sum(latent ** 2, axis=1)
    wsq = jnp.sum(W ** 2, axis=1)
    xb = latent.astype(jnp.bfloat16)
    wb = W.astype(jnp.bfloat16)
    xqst, idx, loss_sum = pl.pallas_call(
        _vq_body,
        grid=(T // _TB,),
        in_specs=[
            pl.BlockSpec((_TB, D), lambda i: (i, 0)),
            pl.BlockSpec((_TB, D), lambda i: (i, 0)),
            pl.BlockSpec((K, D), lambda i: (0, 0)),
            pl.BlockSpec((K, D), lambda i: (0, 0)),
            pl.BlockSpec((_TB,), lambda i: (i,)),
            pl.BlockSpec((K,), lambda i: (0,)),
        ],
        out_specs=[
            pl.BlockSpec((_TB, D), lambda i: (i, 0)),
            pl.BlockSpec((_TB,), lambda i: (i,)),
            pl.BlockSpec((1, 1), lambda i: (0, 0)),
        ],
        out_shape=[
            jax.ShapeDtypeStruct((T, D), jnp.float32),
            jax.ShapeDtypeStruct((T,), jnp.int32),
            jax.ShapeDtypeStruct((1, 1), jnp.float32),
        ],
    )(latent, xb, W, wb, xsq, wsq)
    L = loss_sum[0, 0] / (T * D)
    loss = L + _BETA * L
    return xqst.reshape(x.shape), loss, idx.reshape(B, S)


# speedup vs baseline: 6.7546x; 6.1378x over previous
import jax
import jax.numpy as jnp
from jax.experimental import pallas as pl

_BETA = 0.25
_TB = 256
_WIN = 4096
_KCAND = 4
_FIXC = float(1.5 * 2.0 ** 23 * 2.0 ** -29)


def _exact_d(cand_idx, xbf, wcat, xsq, K):
    TB = xbf.shape[0]
    onehot = (jax.lax.broadcasted_iota(jnp.int32, (TB, K), 1)
              == cand_idx[:, None]).astype(jnp.bfloat16)
    grow = jax.lax.dot_general(onehot, wcat, (((1,), (0,)), ((), ())),
                               preferred_element_type=jnp.float32)
    wrow = grow[:, :32]
    wsq_c = (grow[:, 32] + grow[:, 33]) + grow[:, 34]
    prods = xbf * wrow
    fixc = jnp.float32(_FIXC)
    hi = (prods + fixc) - fixc
    lo = prods - hi
    m = jnp.sum(hi, axis=1) + jnp.sum(lo, axis=1)
    return (xsq + wsq_c) - 2.0 * m


def _window_winner(dwin, base, xbf, wcat, xsq, K):
    cur = dwin
    iota = jax.lax.broadcasted_iota(jnp.int32, dwin.shape, 1)
    best_v = None
    best_i = None
    for _ in range(_KCAND):
        i_r = jnp.argmin(cur, axis=1).astype(jnp.int32)
        cur = jnp.where(iota == i_r[:, None], jnp.float32(jnp.inf), cur)
        gi = i_r + base
        gv = _exact_d(gi, xbf, wcat, xsq, K)
        if best_v is None:
            best_v, best_i = gv, gi
        else:
            better = (gv < best_v) | ((gv == best_v) & (gi < best_i))
            best_v = jnp.where(better, gv, best_v)
            best_i = jnp.where(better, gi, best_i)
    return best_v, best_i


def _vq_body(x_ref, xb_ref, wb_ref, wcat_ref, whi_ref, wmid_ref, wlo_ref,
             xsq_ref, wsq_ref, xqst_ref, idx_ref, loss_ref):
    i = pl.program_id(0)
    K = wb_ref.shape[0]
    x_blk = x_ref[...]
    xb = xb_ref[...]
    wb = wb_ref[...]
    wcat = wcat_ref[...]
    xsq = xsq_ref[...]
    wsq = wsq_ref[...]
    xbf = xb.astype(jnp.float32)
    m = jax.lax.dot_general(xb, wb, (((1,), (1,)), ((), ())),
                            preferred_element_type=jnp.float32)
    d = (xsq[:, None] + wsq[None, :]) - 2.0 * m
    v1, i1 = _window_winner(d[:, :_WIN], 0, xbf, wcat, xsq, K)
    v2, i2 = _window_winner(d[:, _WIN:], _WIN, xbf, wcat, xsq, K)
    v1q = v1.astype(jnp.bfloat16).astype(jnp.float32)
    idx = jnp.where(v2 < v1q, i2, i1)
    onehot = (jax.lax.broadcasted_iota(jnp.int32, (x_blk.shape[0], K), 1)
              == idx[:, None]).astype(jnp.bfloat16)
    q_hi = jax.lax.dot_general(onehot, whi_ref[...], (((1,), (0,)), ((), ())),
                               preferred_element_type=jnp.float32)
    q_mid = jax.lax.dot_general(onehot, wmid_ref[...], (((1,), (0,)), ((), ())),
                                preferred_element_type=jnp.float32)
    q_lo = jax.lax.dot_general(onehot, wlo_ref[...], (((1,), (0,)), ((), ())),
                               preferred_element_type=jnp.float32)
    xq = (q_hi + q_mid) + q_lo
    diff = xq - x_blk
    xqst_ref[...] = x_blk + diff
    idx_ref[...] = idx

    @pl.when(i == 0)
    def _():
        loss_ref[...] = jnp.zeros((1, 1), jnp.float32)

    loss_ref[...] += jnp.sum(diff * diff).reshape(1, 1)


def kernel(x, W):
    B, S, D = x.shape
    T = B * S
    K = W.shape[0]
    latent = x.reshape(T, D)
    xsq = jnp.sum(latent ** 2, axis=1)
    wsq = jnp.sum(W ** 2, axis=1)
    xb = latent.astype(jnp.bfloat16)
    wb = W.astype(jnp.bfloat16)
    w_hi = W.astype(jnp.bfloat16)
    w_r1 = W - w_hi.astype(jnp.float32)
    w_mid = w_r1.astype(jnp.bfloat16)
    w_lo = (w_r1 - w_mid.astype(jnp.float32)).astype(jnp.bfloat16)
    ws_hi = wsq.astype(jnp.bfloat16)
    ws_r1 = wsq - ws_hi.astype(jnp.float32)
    ws_mid = ws_r1.astype(jnp.bfloat16)
    ws_lo = (ws_r1 - ws_mid.astype(jnp.float32)).astype(jnp.bfloat16)
    wcat = jnp.concatenate(
        [wb, ws_hi[:, None], ws_mid[:, None], ws_lo[:, None]], axis=1)
    xqst, idx, loss_sum = pl.pallas_call(
        _vq_body,
        grid=(T // _TB,),
        in_specs=[
            pl.BlockSpec((_TB, D), lambda i: (i, 0)),
            pl.BlockSpec((_TB, D), lambda i: (i, 0)),
            pl.BlockSpec((K, D), lambda i: (0, 0)),
            pl.BlockSpec((K, D + 3), lambda i: (0, 0)),
            pl.BlockSpec((K, D), lambda i: (0, 0)),
            pl.BlockSpec((K, D), lambda i: (0, 0)),
            pl.BlockSpec((K, D), lambda i: (0, 0)),
            pl.BlockSpec((_TB,), lambda i: (i,)),
            pl.BlockSpec((K,), lambda i: (0,)),
        ],
        out_specs=[
            pl.BlockSpec((_TB, D), lambda i: (i, 0)),
            pl.BlockSpec((_TB,), lambda i: (i,)),
            pl.BlockSpec((1, 1), lambda i: (0, 0)),
        ],
        out_shape=[
            jax.ShapeDtypeStruct((T, D), jnp.float32),
            jax.ShapeDtypeStruct((T,), jnp.int32),
            jax.ShapeDtypeStruct((1, 1), jnp.float32),
        ],
    )(latent, xb, wb, wcat, w_hi, w_mid, w_lo, xsq, wsq)
    L = loss_sum[0, 0] / (T * D)
    loss = L + _BETA * L
    return xqst.reshape(x.shape), loss, idx.reshape(B, S)
